# baseline (speedup 1.0000x reference)
import jax
import jax.numpy as jnp
from jax.experimental import pallas as pl

_N, _D, _H = 10000, 128, 128
_BMC = 200
_BM = 400

_DN_T = (((1,), (1,)), ((), ()))
_DN = (((1,), (0,)), ((), ()))


def _cast_kernel(ap_ref, an_ref, op_ref, on_ref):
    op_ref[...] = ap_ref[...].astype(jnp.bfloat16)
    on_ref[...] = an_ref[...].astype(jnp.bfloat16)


def _cast(A_pos, A_neg):
    nb = _N // _BMC
    return pl.pallas_call(
        _cast_kernel,
        grid=(nb,),
        in_specs=[
            pl.BlockSpec((_BMC, _N), lambda i: (i, 0)),
            pl.BlockSpec((_BMC, _N), lambda i: (i, 0)),
        ],
        out_specs=[
            pl.BlockSpec((_BMC, _N), lambda i: (i, 0)),
            pl.BlockSpec((_BMC, _N), lambda i: (i, 0)),
        ],
        out_shape=[
            jax.ShapeDtypeStruct((_N, _N), jnp.bfloat16),
            jax.ShapeDtypeStruct((_N, _N), jnp.bfloat16),
        ],
    )(A_pos, A_neg)


def _h0_kernel(x_ref, w_ref, b_ref, o_ref):
    acc = jax.lax.dot_general(x_ref[...], w_ref[...], _DN_T,
                              preferred_element_type=jnp.float32)
    o_ref[...] = jnp.tanh(acc + b_ref[...])


def _layer_kernel(ap_ref, an_ref, h_ref, wp_ref, wn_ref, bp_ref, bn_ref, o_ref):
    h = h_ref[...]
    hp = jax.lax.dot_general(ap_ref[...], h, _DN, preferred_element_type=jnp.float32)
    hn = jax.lax.dot_general(an_ref[...], h, _DN, preferred_element_type=jnp.float32)
    tp = jax.lax.dot_general(hp.astype(jnp.bfloat16), wp_ref[...], _DN_T,
                             preferred_element_type=jnp.float32) + bp_ref[...]
    tn = jax.lax.dot_general(hn.astype(jnp.bfloat16), wn_ref[...], _DN_T,
                             preferred_element_type=jnp.float32) + bn_ref[...]
    o_ref[...] = jnp.tanh(tp + tn)


def _layer(Ap_bf, An_bf, h_bf, Wp_bf, bp, Wn_bf, bn):
    nb = _N // _BM
    return pl.pallas_call(
        _layer_kernel,
        grid=(nb,),
        in_specs=[
            pl.BlockSpec((_BM, _N), lambda i: (i, 0)),
            pl.BlockSpec((_BM, _N), lambda i: (i, 0)),
            pl.BlockSpec((_N, _H), lambda i: (0, 0)),
            pl.BlockSpec((_H, _H), lambda i: (0, 0)),
            pl.BlockSpec((_H, _H), lambda i: (0, 0)),
            pl.BlockSpec((1, _H), lambda i: (0, 0)),
            pl.BlockSpec((1, _H), lambda i: (0, 0)),
        ],
        out_specs=pl.BlockSpec((_BM, _H), lambda i: (i, 0)),
        out_shape=jax.ShapeDtypeStruct((_N, _H), jnp.float32),
    )(Ap_bf, An_bf, h_bf, Wp_bf, Wn_bf, bp.reshape(1, _H), bn.reshape(1, _H))


def kernel(x, A_pos, A_neg, W_in, b_in, Wp0, bp0, Wn0, bn0, Wp1, bp1, Wn1, bn1):
    bf = jnp.bfloat16
    Ap_bf, An_bf = _cast(A_pos, A_neg)
    h = pl.pallas_call(
        _h0_kernel,
        out_shape=jax.ShapeDtypeStruct((_N, _H), jnp.float32),
    )(x.astype(bf), W_in.astype(bf), b_in.reshape(1, _H))
    h = _layer(Ap_bf, An_bf, h.astype(bf), Wp0.astype(bf), bp0, Wn0.astype(bf), bn0)
    h = _layer(Ap_bf, An_bf, h.astype(bf), Wp1.astype(bf), bp1, Wn1.astype(bf), bn1)
    return h


# speedup vs baseline: 1.3534x; 1.3534x over previous
import jax
import jax.numpy as jnp
from jax.experimental import pallas as pl
from jax.experimental.pallas import tpu as pltpu

_N, _D, _H = 10000, 128, 128
_BM = 200

_DN_T = (((1,), (1,)), ((), ()))
_DN = (((1,), (0,)), ((), ()))


def _propagate(h, ap_ref, an_ref, wp_ref, wn_ref, bp_ref, bn_ref, o_ref):
    hp = jax.lax.dot_general(ap_ref[...], h, _DN,
                             preferred_element_type=jnp.float32)
    hn = jax.lax.dot_general(an_ref[...], h, _DN,
                             preferred_element_type=jnp.float32)
    tp = jax.lax.dot_general(hp, wp_ref[...], _DN_T,
                             preferred_element_type=jnp.float32) + bp_ref[...]
    tn = jax.lax.dot_general(hn, wn_ref[...], _DN_T,
                             preferred_element_type=jnp.float32) + bn_ref[...]
    o_ref[...] = jnp.tanh(tp + tn)


def _layer1_kernel(x_ref, win_ref, bin_ref, ap_ref, an_ref, wp_ref, wn_ref,
                   bp_ref, bn_ref, o_ref, h_ref):
    @pl.when(pl.program_id(0) == 0)
    def _():
        acc = jax.lax.dot_general(x_ref[...], win_ref[...], _DN_T,
                                  preferred_element_type=jnp.float32)
        h_ref[...] = jnp.tanh(acc + bin_ref[...])

    _propagate(h_ref[...], ap_ref, an_ref, wp_ref, wn_ref, bp_ref, bn_ref, o_ref)


def _layer2_kernel(h_in_ref, ap_ref, an_ref, wp_ref, wn_ref, bp_ref, bn_ref, o_ref):
    _propagate(h_in_ref[...], ap_ref, an_ref, wp_ref, wn_ref, bp_ref, bn_ref, o_ref)


def _strip_specs():
    return [
        pl.BlockSpec((_BM, _N), lambda i: (i, 0)),
        pl.BlockSpec((_BM, _N), lambda i: (i, 0)),
        pl.BlockSpec((_H, _H), lambda i: (0, 0)),
        pl.BlockSpec((_H, _H), lambda i: (0, 0)),
        pl.BlockSpec((1, _H), lambda i: (0, 0)),
        pl.BlockSpec((1, _H), lambda i: (0, 0)),
    ]


def kernel(x, A_pos, A_neg, W_in, b_in, Wp0, bp0, Wn0, bn0, Wp1, bp1, Wn1, bn1):
    nb = _N // _BM
    h = pl.pallas_call(
        _layer1_kernel,
        grid=(nb,),
        in_specs=[
            pl.BlockSpec((_N, _D), lambda i: (0, 0)),
            pl.BlockSpec((_H, _D), lambda i: (0, 0)),
            pl.BlockSpec((1, _H), lambda i: (0, 0)),
        ] + _strip_specs(),
        out_specs=pl.BlockSpec((_BM, _H), lambda i: (i, 0)),
        out_shape=jax.ShapeDtypeStruct((_N, _H), jnp.float32),
        scratch_shapes=[pltpu.VMEM((_N, _H), jnp.float32)],
    )(x, W_in, b_in.reshape(1, _H), A_pos, A_neg, Wp0, Wn0,
      bp0.reshape(1, _H), bn0.reshape(1, _H))
    h = pl.pallas_call(
        _layer2_kernel,
        grid=(nb,),
        in_specs=[pl.BlockSpec((_N, _H), lambda i: (0, 0))] + _strip_specs(),
        out_specs=pl.BlockSpec((_BM, _H), lambda i: (i, 0)),
        out_shape=jax.ShapeDtypeStruct((_N, _H), jnp.float32),
    )(h, A_pos, A_neg, Wp1, Wn1, bp1.reshape(1, _H), bn1.reshape(1, _H))
    return h


# speedup vs baseline: 1.3712x; 1.0131x over previous
import jax
import jax.numpy as jnp
from jax.experimental import pallas as pl
from jax.experimental.pallas import tpu as pltpu

_N, _D, _H = 10000, 128, 128
_BM = 200
_NB = _N // _BM

_DN_T = (((1,), (1,)), ((), ()))
_DN = (((1,), (0,)), ((), ()))


def _kernel(x_ref, win_ref, bin_ref, ap_ref, an_ref, wp0_ref, wn0_ref,
            bp0_ref, bn0_ref, wp1_ref, wn1_ref, bp1_ref, bn1_ref,
            o_ref, hsc_ref):
    i = pl.program_id(0)

    @pl.when(i == 0)
    def _():
        acc = jax.lax.dot_general(x_ref[...], win_ref[...], _DN_T,
                                  preferred_element_type=jnp.float32)
        hsc_ref[pl.ds(0, _N), :] = jnp.tanh(acc + bin_ref[...])

    lyr = i // _NB
    in_l1 = i < _NB
    h = hsc_ref[pl.ds(lyr * _N, _N), :]
    wp = jnp.where(in_l1, wp0_ref[...], wp1_ref[...])
    wn = jnp.where(in_l1, wn0_ref[...], wn1_ref[...])
    bp = jnp.where(in_l1, bp0_ref[...], bp1_ref[...])
    bn = jnp.where(in_l1, bn0_ref[...], bn1_ref[...])

    hp = jax.lax.dot_general(ap_ref[...], h, _DN,
                             preferred_element_type=jnp.float32)
    hn = jax.lax.dot_general(an_ref[...], h, _DN,
                             preferred_element_type=jnp.float32)
    tp = jax.lax.dot_general(hp, wp, _DN_T,
                             preferred_element_type=jnp.float32) + bp
    tn = jax.lax.dot_general(hn, wn, _DN_T,
                             preferred_element_type=jnp.float32) + bn
    out = jnp.tanh(tp + tn)

    @pl.when(in_l1)
    def _():
        hsc_ref[pl.ds(_N + (i % _NB) * _BM, _BM), :] = out

    o_ref[...] = out


def kernel(x, A_pos, A_neg, W_in, b_in, Wp0, bp0, Wn0, bn0, Wp1, bp1, Wn1, bn1):
    const = lambda spec: pl.BlockSpec(spec, lambda i: (0, 0))
    strip = pl.BlockSpec((_BM, _N), lambda i: (i % _NB, 0))
    return pl.pallas_call(
        _kernel,
        grid=(2 * _NB,),
        in_specs=[
            const((_N, _D)),
            const((_H, _D)),
            const((1, _H)),
            strip,
            strip,
            const((_H, _H)),
            const((_H, _H)),
            const((1, _H)),
            const((1, _H)),
            const((_H, _H)),
            const((_H, _H)),
            const((1, _H)),
            const((1, _H)),
        ],
        out_specs=pl.BlockSpec((_BM, _H), lambda i: (i % _NB, 0)),
        out_shape=jax.ShapeDtypeStruct((_N, _H), jnp.float32),
        scratch_shapes=[pltpu.VMEM((2 * _N, _H), jnp.float32)],
    )(x, W_in, b_in.reshape(1, _H), A_pos, A_neg,
      Wp0, Wn0, bp0.reshape(1, _H), bn0.reshape(1, _H),
      Wp1, Wn1, bp1.reshape(1, _H), bn1.reshape(1, _H))
